# baseline (speedup 1.0000x reference)
import functools

import jax
import jax.numpy as jnp
from jax import lax
from jax.experimental import pallas as pl
from jax.experimental.pallas import tpu as pltpu
from jax.experimental.pallas import tpu_sc as plsc

_LANES = 16


def _weights_body(x_ref, w_ref, p_ref):
    logits = jax.lax.dot_general(
        x_ref[...], w_ref[...], (((1,), (1,)), ((), ())),
        preferred_element_type=jnp.float32)
    m = jnp.max(logits, axis=1, keepdims=True)
    e = jnp.exp(logits - m)
    p_ref[...] = e / jnp.sum(e, axis=1, keepdims=True)


def _gate_weights(x, W):
    B, D = x.shape
    NUM = W.shape[0]
    BLK = 1024
    return pl.pallas_call(
        _weights_body,
        grid=(B // BLK,),
        in_specs=[
            pl.BlockSpec((BLK, D), lambda i: (i, 0)),
            pl.BlockSpec((NUM, D), lambda i: (0, 0)),
        ],
        out_specs=pl.BlockSpec((BLK, NUM), lambda i: (i, 0)),
        out_shape=jax.ShapeDtypeStruct((B, NUM), jnp.float32),
    )(x, W)


def _sc_pool(p_flat, experts, T=4):
    B, NUM, D = experts.shape
    NC, NS = 2, 16
    NW = NC * NS
    bpw = B // NW
    nchunks = bpw // T
    mesh = plsc.VectorSubcoreMesh(core_axis_name="c", subcore_axis_name="s")

    @functools.partial(
        pl.kernel,
        out_type=jax.ShapeDtypeStruct((B, D), jnp.float32),
        mesh=mesh,
        scratch_types=[
            pltpu.VMEM((bpw * NUM,), jnp.float32),
            pltpu.VMEM((T, NUM, D), jnp.float32),
            pltpu.VMEM((T, NUM, D), jnp.float32),
            pltpu.VMEM((T, D), jnp.float32),
            pltpu.SemaphoreType.DMA,
            pltpu.SemaphoreType.DMA,
        ],
    )
    def k(p_hbm, e_hbm, o_hbm, p_v, ea, eb, o_v, sa, sb):
        wid = lax.axis_index("s") * NC + lax.axis_index("c")
        base = wid * bpw
        pltpu.sync_copy(p_hbm.at[pl.ds(base * NUM, bpw * NUM)], p_v)

        bufs = ((ea, sa), (eb, sb))

        for b in range(2):
            ebuf, sem = bufs[b]
            pltpu.async_copy(e_hbm.at[pl.ds(base + b * T, T)], ebuf, sem)

        def compute_chunk(c, ebuf):
            for tp in range(T // 2):
                wvec = p_v[pl.ds((c * T + tp * 2) * NUM, _LANES)]
                for half in range(2):
                    t = tp * 2 + half
                    w = [wvec[half * NUM + n] for n in range(NUM)]

                    def dbody(d, carry, t=t, w=w):
                        sl = pl.ds(d * _LANES, _LANES)
                        acc = w[0] * ebuf[t, 0, sl]
                        for n in range(1, NUM):
                            acc = acc + w[n] * ebuf[t, n, sl]
                        o_v[t, sl] = acc
                        return carry

                    lax.fori_loop(0, D // _LANES, dbody, 0)

        def pair_body(i, carry):
            for b in range(2):
                ebuf, sem = bufs[b]
                c = i * 2 + b
                tok0 = base + c * T
                pltpu.make_async_copy(
                    e_hbm.at[pl.ds(tok0, T)], ebuf, sem).wait()
                compute_chunk(c, ebuf)
                pltpu.sync_copy(o_v, o_hbm.at[pl.ds(tok0, T)])

                @pl.when(c + 2 < nchunks)
                def _prefetch():
                    pltpu.async_copy(
                        e_hbm.at[pl.ds(tok0 + 2 * T, T)], ebuf, sem)

            return carry

        lax.fori_loop(0, nchunks // 2, pair_body, 0)

    return k(p_flat, experts)


@jax.jit
def kernel(x, experts, W):
    B, D = x.shape
    NUM = W.shape[0]
    p = _gate_weights(x, W)
    return _sc_pool(p.reshape(B * NUM), experts)


# speedup vs baseline: 1.7076x; 1.7076x over previous
import functools

import jax
import jax.numpy as jnp
from jax import lax
from jax.experimental import pallas as pl
from jax.experimental.pallas import tpu as pltpu
from jax.experimental.pallas import tpu_sc as plsc

_LANES = 16
_BLK = 256
_S_TC = 5376


def _softmax_rows(logits):
    m = jnp.max(logits, axis=1, keepdims=True)
    e = jnp.exp(logits - m)
    return e / jnp.sum(e, axis=1, keepdims=True)


def _fused_body(x_ref, w_ref, e_ref, o_ref):
    logits = jax.lax.dot_general(
        x_ref[...], w_ref[...], (((1,), (1,)), ((), ())),
        preferred_element_type=jnp.float32)
    p = _softmax_rows(logits)
    num = e_ref.shape[1]
    acc = p[:, 0:1] * e_ref[:, 0, :]
    for n in range(1, num):
        acc = acc + p[:, n:n + 1] * e_ref[:, n, :]
    o_ref[...] = acc


def _tc_fused(x, experts, W, S):
    B, D = x.shape
    NUM = W.shape[0]
    return pl.pallas_call(
        _fused_body,
        grid=(S // _BLK,),
        in_specs=[
            pl.BlockSpec((_BLK, D), lambda i: (i, 0)),
            pl.BlockSpec((NUM, D), lambda i: (0, 0)),
            pl.BlockSpec((_BLK, NUM, D), lambda i: (i, 0, 0)),
        ],
        out_specs=pl.BlockSpec((_BLK, D), lambda i: (i, 0)),
        out_shape=jax.ShapeDtypeStruct((S, D), jnp.float32),
    )(x, W, experts)


def _weights_body(x_ref, w_ref, p_ref):
    logits = jax.lax.dot_general(
        x_ref[...], w_ref[...], (((1,), (1,)), ((), ())),
        preferred_element_type=jnp.float32)
    p_ref[...] = _softmax_rows(logits)


def _gate_weights(x, W, row0, nrows):
    B, D = x.shape
    NUM = W.shape[0]
    off = row0 // _BLK
    return pl.pallas_call(
        _weights_body,
        grid=(nrows // _BLK,),
        in_specs=[
            pl.BlockSpec((_BLK, D), lambda i: (i + off, 0)),
            pl.BlockSpec((NUM, D), lambda i: (0, 0)),
        ],
        out_specs=pl.BlockSpec((_BLK, NUM), lambda i: (i, 0)),
        out_shape=jax.ShapeDtypeStruct((nrows, NUM), jnp.float32),
    )(x, W)


def _sc_pool(p_flat, experts, row0, nrows, T=4):
    B, NUM, D = experts.shape
    NC, NS = 2, 16
    NW = NC * NS
    bpw = nrows // NW
    nchunks = bpw // T
    mesh = plsc.VectorSubcoreMesh(core_axis_name="c", subcore_axis_name="s")

    @functools.partial(
        pl.kernel,
        out_type=jax.ShapeDtypeStruct((nrows, D), jnp.float32),
        mesh=mesh,
        scratch_types=[
            pltpu.VMEM((bpw * NUM,), jnp.float32),
            pltpu.VMEM((T, NUM, D), jnp.float32),
            pltpu.VMEM((T, NUM, D), jnp.float32),
            pltpu.VMEM((T, D), jnp.float32),
            pltpu.SemaphoreType.DMA,
            pltpu.SemaphoreType.DMA,
        ],
    )
    def k(p_hbm, e_hbm, o_hbm, p_v, ea, eb, o_v, sa, sb):
        wid = lax.axis_index("s") * NC + lax.axis_index("c")
        lbase = wid * bpw
        gbase = row0 + lbase
        pltpu.sync_copy(p_hbm.at[pl.ds(lbase * NUM, bpw * NUM)], p_v)

        bufs = ((ea, sa), (eb, sb))

        for b in range(2):
            ebuf, sem = bufs[b]
            pltpu.async_copy(e_hbm.at[pl.ds(gbase + b * T, T)], ebuf, sem)

        def compute_chunk(c, ebuf):
            for tp in range(T // 2):
                wvec = p_v[pl.ds((c * T + tp * 2) * NUM, _LANES)]
                for half in range(2):
                    t = tp * 2 + half
                    w = [wvec[half * NUM + n] for n in range(NUM)]

                    def dbody(d, carry, t=t, w=w):
                        sl = pl.ds(d * _LANES, _LANES)
                        acc = w[0] * ebuf[t, 0, sl]
                        for n in range(1, NUM):
                            acc = acc + w[n] * ebuf[t, n, sl]
                        o_v[t, sl] = acc
                        return carry

                    lax.fori_loop(0, D // _LANES, dbody, 0, unroll=4)

        def pair_body(i, carry):
            for b in range(2):
                ebuf, sem = bufs[b]
                c = i * 2 + b
                pltpu.make_async_copy(
                    e_hbm.at[pl.ds(gbase + c * T, T)], ebuf, sem).wait()
                compute_chunk(c, ebuf)
                pltpu.sync_copy(o_v, o_hbm.at[pl.ds(lbase + c * T, T)])

                @pl.when(c + 2 < nchunks)
                def _prefetch():
                    pltpu.async_copy(
                        e_hbm.at[pl.ds(gbase + (c + 2) * T, T)], ebuf, sem)

            return carry

        lax.fori_loop(0, nchunks // 2, pair_body, 0)

    return k(p_flat, experts)


@jax.jit
def kernel(x, experts, W):
    B, D = x.shape
    NUM = W.shape[0]
    n_sc = B - _S_TC
    p_sc = _gate_weights(x, W, _S_TC, n_sc)
    out_sc = _sc_pool(p_sc.reshape(n_sc * NUM), experts, _S_TC, n_sc)
    out_tc = _tc_fused(x, experts, W, _S_TC)
    return jnp.concatenate([out_tc, out_sc], axis=0)
